# baseline (speedup 1.0000x reference)
import functools
import numpy as np
import jax
import jax.numpy as jnp
from jax.experimental import pallas as pl

_EPS = 1e-5


def _cdiv(a, b):
    return -(-a // b)


def _rup(a, b):
    return _cdiv(a, b) * b


def _conv_bn_body(x_ref, w_ref, y_ref, st_ref, *, taps, BM, C, Wpx, Wvalid, Mout):
    n = pl.program_id(0)
    b = pl.program_id(1)
    base = b * BM
    co = w_ref.shape[2]
    acc = jnp.zeros((BM, co), jnp.float32)
    for t, (ph, off) in enumerate(taps):
        g8, s = off & ~7, off & 7
        xs = x_ref[0, pl.ds(base + g8, BM + 8), pl.ds(ph * C, C)][s:s + BM]
        acc = acc + jnp.dot(xs, w_ref[t], preferred_element_type=jnp.float32)
    y_ref[0] = acc
    pidx = base + jax.lax.broadcasted_iota(jnp.int32, (BM, 1), 0)
    valid = ((pidx % Wpx) < Wvalid) & (pidx < Mout)
    m = valid.astype(jnp.float32)
    s1 = jnp.sum(acc * m, axis=0, keepdims=True)
    s2 = jnp.sum(acc * acc * m, axis=0, keepdims=True)

    @pl.when((n == 0) & (b == 0))
    def _():
        st_ref[...] = jnp.zeros_like(st_ref)

    st_ref[0:1, :] += s1
    st_ref[1:2, :] += s2


def _conv_sig_body(x_ref, w_ref, bias_ref, y_ref, *, taps, BM, C):
    b = pl.program_id(1)
    base = b * BM
    co = w_ref.shape[2]
    acc = jnp.zeros((BM, co), jnp.float32)
    for t, (ph, off) in enumerate(taps):
        g8, s = off & ~7, off & 7
        xs = x_ref[0, pl.ds(base + g8, BM + 8), pl.ds(ph * C, C)][s:s + BM]
        acc = acc + jnp.dot(xs, w_ref[t], preferred_element_type=jnp.float32)
    y_ref[0] = jax.nn.sigmoid(acc + bias_ref[0:1, :])


def _bn_relu_framed_body(y_ref, ab_ref, t_ref, *, BM, Wpx, Wv, Mout):
    b = pl.program_id(1)
    val = jnp.maximum(y_ref[0] * ab_ref[0:1, :] + ab_ref[1:2, :], 0.0)
    pidx = (b - 1) * BM + jax.lax.broadcasted_iota(jnp.int32, (BM, 1), 0)
    ok = (pidx >= 0) & (pidx < Mout) & ((pidx % Wpx) < Wv)
    t_ref[0] = jnp.where(ok, val, 0.0)


def _conv_call(xf, w9, nblk, BM, body):
    N, R, L = xf.shape
    C, Co = w9.shape[1], w9.shape[2]
    MP = nblk * BM
    return pl.pallas_call(
        body,
        grid=(N, nblk),
        in_specs=[
            pl.BlockSpec((1, R, L), lambda n, b: (n, 0, 0)),
            pl.BlockSpec((9, C, Co), lambda n, b: (0, 0, 0)),
        ],
        out_specs=[
            pl.BlockSpec((1, BM, Co), lambda n, b: (n, b, 0)),
            pl.BlockSpec((8, Co), lambda n, b: (0, 0)),
        ],
        out_shape=[
            jax.ShapeDtypeStruct((N, MP, Co), jnp.float32),
            jax.ShapeDtypeStruct((8, Co), jnp.float32),
        ],
    )(xf, w9)


def _enc_conv(act, w9, BM):
    N, H, W, C = act.shape
    Hp, Wp = H + 2, W + 2
    H2, W2 = Hp // 2, Wp // 2
    Ho, Wo = H // 2, W // 2
    Mout = Ho * W2
    BM = min(BM, _rup(Mout, 8))
    nblk = _cdiv(Mout, BM)
    R = nblk * BM + W2 + 9
    xp = jnp.pad(act, ((0, 0), (1, 1), (1, 1), (0, 0)))
    ph = xp.reshape(N, H2, 2, W2, 2, C).transpose(0, 1, 3, 2, 4, 5)
    ph = ph.reshape(N, H2 * W2, 4 * C)
    ph = jnp.pad(ph, ((0, 0), (0, R - H2 * W2), (0, 0)))
    taps = [((dy % 2) * 2 + (dx % 2), (dy // 2) * W2 + (dx // 2))
            for dy in range(3) for dx in range(3)]
    body = functools.partial(_conv_bn_body, taps=taps, BM=BM, C=C, Wpx=W2,
                             Wvalid=Wo, Mout=Mout)
    y, st = _conv_call(ph, w9, nblk, BM, body)
    return y, st, (Ho, W2, Wo, BM, nblk)


def _framed_conv(t, w9, geom, sig_bias=None):
    N = t.shape[0]
    H, Wpx, Wv, BM, nblk = geom
    C = w9.shape[1]
    Mout = H * Wpx
    taps = [(0, BM + (dy - 1) * Wpx + dx - 1) for dy in range(3)
            for dx in range(3)]
    if sig_bias is None:
        body = functools.partial(_conv_bn_body, taps=taps, BM=BM, C=C,
                                 Wpx=Wpx, Wvalid=Wv, Mout=Mout)
        y, st = _conv_call(t, w9, nblk, BM, body)
        return y, st, (H, Wpx, Wv, BM, nblk)
    body = functools.partial(_conv_sig_body, taps=taps, BM=BM, C=C)
    Co = w9.shape[2]
    R = t.shape[1]
    y = pl.pallas_call(
        body,
        grid=(N, nblk),
        in_specs=[
            pl.BlockSpec((1, R, C), lambda n, b: (n, 0, 0)),
            pl.BlockSpec((9, C, Co), lambda n, b: (0, 0, 0)),
            pl.BlockSpec((8, Co), lambda n, b: (0, 0)),
        ],
        out_specs=pl.BlockSpec((1, BM, Co), lambda n, b: (n, b, 0)),
        out_shape=jax.ShapeDtypeStruct((N, nblk * BM, Co), jnp.float32),
    )(t, w9, sig_bias)
    return y, None, (H, Wpx, Wv, BM, nblk)


def _bn_relu_framed(y, st, gamma, beta, cnt, geom):
    N, MP, Co = y.shape
    H, Wpx, Wv, BM, nblk = geom
    Mout = H * Wpx
    s1, s2 = st[0], st[1]
    mean = s1 / cnt
    var = s2 / cnt - mean * mean
    a = gamma * jax.lax.rsqrt(var + _EPS)
    b = beta - mean * a
    ab = jnp.zeros((8, Co), jnp.float32).at[0].set(a).at[1].set(b)
    body = functools.partial(_bn_relu_framed_body, BM=BM, Wpx=Wpx, Wv=Wv,
                             Mout=Mout)
    return pl.pallas_call(
        body,
        grid=(N, nblk + 2),
        in_specs=[
            pl.BlockSpec((1, BM, Co),
                         lambda n, b_: (n, jnp.clip(b_ - 1, 0, nblk - 1), 0)),
            pl.BlockSpec((8, Co), lambda n, b_: (0, 0)),
        ],
        out_specs=pl.BlockSpec((1, BM, Co), lambda n, b_: (n, b_, 0)),
        out_shape=jax.ShapeDtypeStruct((N, (nblk + 2) * BM, Co), jnp.float32),
    )(y, ab)


def _up_to_framed(t, geom, BMmax):
    N, _, C = t.shape
    H, Wpx_s, Wv_s, BMs, _ = geom
    W = Wv_s
    P0s = BMs
    H2, W2 = 2 * H, 2 * W
    Wp2 = W2 + 2
    Mout2 = H2 * Wp2
    BM2 = min(BMmax, _rup(Mout2, 8))
    nblk2 = _cdiv(Mout2, BM2)
    P0t = BM2
    R2 = (nblk2 + 2) * BM2
    r = np.arange(R2, dtype=np.int64)
    q = r - P0t
    h2 = np.clip(q // Wp2, 0, H2 - 1)
    w2 = np.minimum(np.abs(q) % Wp2, W2 - 1)
    valid = (q >= 0) & (q < Mout2) & ((q % Wp2) < W2)
    ph = h2 * ((H - 1) / (H2 - 1))
    i0 = np.floor(ph).astype(np.int64)
    fh = (ph - i0).astype(np.float32)
    i1 = np.minimum(i0 + 1, H - 1)
    pw = w2 * ((W - 1) / (W2 - 1))
    j0 = np.floor(pw).astype(np.int64)
    fw = (pw - j0).astype(np.float32)
    j1 = np.minimum(j0 + 1, W - 1)

    def fl(i, j):
        return jnp.asarray((P0s + i * Wpx_s + j).astype(np.int32))

    v = valid.astype(np.float32)

    def wt(a_, b_):
        return jnp.asarray((a_ * b_ * v).astype(np.float32))[None, :, None]

    out = (jnp.take(t, fl(i0, j0), axis=1) * wt(1 - fh, 1 - fw)
           + jnp.take(t, fl(i0, j1), axis=1) * wt(1 - fh, fw)
           + jnp.take(t, fl(i1, j0), axis=1) * wt(fh, 1 - fw)
           + jnp.take(t, fl(i1, j1), axis=1) * wt(fh, fw))
    return out, (H2, Wp2, W2, BM2, nblk2)


def _extract(t, geom):
    N, _, Co = t.shape
    Ho, Wpx, Wv, BM, _ = geom
    return t[:, BM:BM + Ho * Wpx].reshape(N, Ho, Wpx, Co)[:, :, :Wv]


def _w9(W):
    return jnp.transpose(W, (2, 3, 1, 0)).reshape(9, W.shape[1], W.shape[0])


@jax.jit
def kernel(x, W_enc1, g_enc1, b_enc1, W_enc2, g_enc2, b_enc2, W_enc3, g_enc3,
           b_enc3, W_enc4, g_enc4, b_enc4, W_dec1, g_dec1, b_dec1, W_dec2,
           g_dec2, b_dec2, W_dec3, g_dec3, b_dec3, W_dec4, g_dec4, b_dec4,
           W_out, b_out):
    N = x.shape[0]
    BM = 2048
    act = jnp.transpose(x, (0, 2, 3, 1))
    act = jnp.pad(act, ((0, 0), (0, 0), (0, 0), (0, 7)))
    enc = [(W_enc1, g_enc1, b_enc1), (W_enc2, g_enc2, b_enc2),
           (W_enc3, g_enc3, b_enc3), (W_enc4, g_enc4, b_enc4)]
    t = None
    for k, (Wc, g_, be_) in enumerate(enc):
        w9 = _w9(Wc)
        if k == 0:
            w9 = jnp.pad(w9, ((0, 0), (0, 7), (0, 0)))
        y, st, geom = _enc_conv(act, w9, BM)
        cnt = jnp.float32(N * geom[0] * geom[2])
        t = _bn_relu_framed(y, st, g_, be_, cnt, geom)
        if k < 3:
            act = _extract(t, geom)
    latent = jnp.transpose(_extract(t, geom), (0, 3, 1, 2))
    dec = [(W_dec1, g_dec1, b_dec1), (W_dec2, g_dec2, b_dec2),
           (W_dec3, g_dec3, b_dec3), (W_dec4, g_dec4, b_dec4)]
    for Wc, g_, be_ in dec:
        up, geom = _up_to_framed(t, geom, BM)
        y, st, geom = _framed_conv(up, _w9(Wc), geom)
        cnt = jnp.float32(N * geom[0] * geom[2])
        t = _bn_relu_framed(y, st, g_, be_, cnt, geom)
    w9o = jnp.pad(_w9(W_out), ((0, 0), (0, 0), (0, 7)))
    bias = jnp.broadcast_to(b_out[0], (8, 8)).astype(jnp.float32)
    y, _, geom = _framed_conv(t, w9o, geom, sig_bias=bias)
    Ho, Wpx, Wv, _, _ = geom
    out = y[:, :Ho * Wpx].reshape(N, Ho, Wpx, 8)[:, :, :Wv, 0:1]
    out = jnp.transpose(out, (0, 3, 1, 2))
    return (out, latent)


# speedup vs baseline: 2.5145x; 2.5145x over previous
import functools
import numpy as np
import jax
import jax.numpy as jnp
from jax.experimental import pallas as pl

_EPS = 1e-5


def _cdiv(a, b):
    return -(-a // b)


def _rup(a, b):
    return _cdiv(a, b) * b


def _conv_bn_body(x_ref, w_ref, y_ref, st_ref, *, taps, BM, C, Wpx, Wvalid, Mout):
    n = pl.program_id(0)
    b = pl.program_id(1)
    base = b * BM
    co = w_ref.shape[2]
    acc = jnp.zeros((BM, co), jnp.float32)
    for t, (ph, off) in enumerate(taps):
        g8, s = off & ~7, off & 7
        xs = x_ref[0, pl.ds(base + g8, BM + 8), pl.ds(ph * C, C)][s:s + BM]
        acc = acc + jnp.dot(xs, w_ref[t], preferred_element_type=jnp.float32)
    y_ref[0] = acc
    pidx = base + jax.lax.broadcasted_iota(jnp.int32, (BM, 1), 0)
    valid = ((pidx % Wpx) < Wvalid) & (pidx < Mout)
    m = valid.astype(jnp.float32)
    s1 = jnp.sum(acc * m, axis=0, keepdims=True)
    s2 = jnp.sum(acc * acc * m, axis=0, keepdims=True)

    @pl.when((n == 0) & (b == 0))
    def _():
        st_ref[...] = jnp.zeros_like(st_ref)

    st_ref[0:1, :] += s1
    st_ref[1:2, :] += s2


def _conv_sig_body(x_ref, w_ref, bias_ref, y_ref, *, taps, BM, C):
    b = pl.program_id(1)
    base = b * BM
    co = w_ref.shape[2]
    acc = jnp.zeros((BM, co), jnp.float32)
    for t, (ph, off) in enumerate(taps):
        g8, s = off & ~7, off & 7
        xs = x_ref[0, pl.ds(base + g8, BM + 8), pl.ds(ph * C, C)][s:s + BM]
        acc = acc + jnp.dot(xs, w_ref[t], preferred_element_type=jnp.float32)
    y_ref[0] = jax.nn.sigmoid(acc + bias_ref[0:1, :])


def _bn_relu_framed_body(y_ref, ab_ref, t_ref, *, BM, Wpx, Wv, Mout):
    b = pl.program_id(1)
    val = jnp.maximum(y_ref[0] * ab_ref[0:1, :] + ab_ref[1:2, :], 0.0)
    pidx = (b - 1) * BM + jax.lax.broadcasted_iota(jnp.int32, (BM, 1), 0)
    ok = (pidx >= 0) & (pidx < Mout) & ((pidx % Wpx) < Wv)
    t_ref[0] = jnp.where(ok, val, 0.0)


def _conv_call(xf, w9, nblk, BM, body):
    N, R, L = xf.shape
    C, Co = w9.shape[1], w9.shape[2]
    MP = nblk * BM
    return pl.pallas_call(
        body,
        grid=(N, nblk),
        in_specs=[
            pl.BlockSpec((1, R, L), lambda n, b: (n, 0, 0)),
            pl.BlockSpec((9, C, Co), lambda n, b: (0, 0, 0)),
        ],
        out_specs=[
            pl.BlockSpec((1, BM, Co), lambda n, b: (n, b, 0)),
            pl.BlockSpec((8, Co), lambda n, b: (0, 0)),
        ],
        out_shape=[
            jax.ShapeDtypeStruct((N, MP, Co), jnp.float32),
            jax.ShapeDtypeStruct((8, Co), jnp.float32),
        ],
    )(xf, w9)


def _enc_conv(act, w9, BM):
    N, H, W, C = act.shape
    Hp, Wp = H + 2, W + 2
    H2, W2 = Hp // 2, Wp // 2
    Ho, Wo = H // 2, W // 2
    Mout = Ho * W2
    BM = min(BM, _rup(Mout, 8))
    nblk = _cdiv(Mout, BM)
    R = nblk * BM + W2 + 9
    xp = jnp.pad(act, ((0, 0), (1, 1), (1, 1), (0, 0)))
    ph = xp.reshape(N, H2, 2, W2, 2, C).transpose(0, 1, 3, 2, 4, 5)
    ph = ph.reshape(N, H2 * W2, 4 * C)
    ph = jnp.pad(ph, ((0, 0), (0, R - H2 * W2), (0, 0)))
    taps = [((dy % 2) * 2 + (dx % 2), (dy // 2) * W2 + (dx // 2))
            for dy in range(3) for dx in range(3)]
    body = functools.partial(_conv_bn_body, taps=taps, BM=BM, C=C, Wpx=W2,
                             Wvalid=Wo, Mout=Mout)
    y, st = _conv_call(ph, w9, nblk, BM, body)
    return y, st, (Ho, W2, Wo, BM, nblk)


def _framed_conv(t, w9, geom, sig_bias=None):
    N = t.shape[0]
    H, Wpx, Wv, BM, nblk = geom
    C = w9.shape[1]
    Mout = H * Wpx
    taps = [(0, BM + (dy - 1) * Wpx + dx - 1) for dy in range(3)
            for dx in range(3)]
    if sig_bias is None:
        body = functools.partial(_conv_bn_body, taps=taps, BM=BM, C=C,
                                 Wpx=Wpx, Wvalid=Wv, Mout=Mout)
        y, st = _conv_call(t, w9, nblk, BM, body)
        return y, st, (H, Wpx, Wv, BM, nblk)
    body = functools.partial(_conv_sig_body, taps=taps, BM=BM, C=C)
    Co = w9.shape[2]
    R = t.shape[1]
    y = pl.pallas_call(
        body,
        grid=(N, nblk),
        in_specs=[
            pl.BlockSpec((1, R, C), lambda n, b: (n, 0, 0)),
            pl.BlockSpec((9, C, Co), lambda n, b: (0, 0, 0)),
            pl.BlockSpec((8, Co), lambda n, b: (0, 0)),
        ],
        out_specs=pl.BlockSpec((1, BM, Co), lambda n, b: (n, b, 0)),
        out_shape=jax.ShapeDtypeStruct((N, nblk * BM, Co), jnp.float32),
    )(t, w9, sig_bias)
    return y, None, (H, Wpx, Wv, BM, nblk)


def _bn_relu_framed(y, st, gamma, beta, cnt, geom):
    N, MP, Co = y.shape
    H, Wpx, Wv, BM, nblk = geom
    Mout = H * Wpx
    s1, s2 = st[0], st[1]
    mean = s1 / cnt
    var = s2 / cnt - mean * mean
    a = gamma * jax.lax.rsqrt(var + _EPS)
    b = beta - mean * a
    ab = jnp.zeros((8, Co), jnp.float32).at[0].set(a).at[1].set(b)
    body = functools.partial(_bn_relu_framed_body, BM=BM, Wpx=Wpx, Wv=Wv,
                             Mout=Mout)
    return pl.pallas_call(
        body,
        grid=(N, nblk + 2),
        in_specs=[
            pl.BlockSpec((1, BM, Co),
                         lambda n, b_: (n, jnp.clip(b_ - 1, 0, nblk - 1), 0)),
            pl.BlockSpec((8, Co), lambda n, b_: (0, 0)),
        ],
        out_specs=pl.BlockSpec((1, BM, Co), lambda n, b_: (n, b_, 0)),
        out_shape=jax.ShapeDtypeStruct((N, (nblk + 2) * BM, Co), jnp.float32),
    )(y, ab)


def _up_to_framed(t, geom, BMmax):
    N, _, C = t.shape
    x = _extract(t, geom)
    H, W = x.shape[1], x.shape[2]

    def idx(s):
        out = 2 * s
        pos = np.arange(out, dtype=np.float64) * ((s - 1) / (out - 1))
        i0 = np.floor(pos).astype(np.int32)
        i1 = np.minimum(i0 + 1, s - 1)
        f = (pos - i0).astype(np.float32)
        return i0, i1, f

    i0, i1, f = idx(H)
    x = x[:, i0] * (1.0 - f)[None, :, None, None] + x[:, i1] * f[None, :, None, None]
    j0, j1, g = idx(W)
    x = x[:, :, j0] * (1.0 - g)[None, None, :, None] + x[:, :, j1] * g[None, None, :, None]
    H2, W2 = 2 * H, 2 * W
    Wp2 = W2 + 2
    Mout2 = H2 * Wp2
    BM2 = min(BMmax, _rup(Mout2, 8))
    nblk2 = _cdiv(Mout2, BM2)
    R2 = (nblk2 + 2) * BM2
    up = jnp.pad(x, ((0, 0), (0, 0), (0, 2), (0, 0))).reshape(N, Mout2, C)
    up = jnp.pad(up, ((0, 0), (BM2, R2 - BM2 - Mout2), (0, 0)))
    return up, (H2, Wp2, W2, BM2, nblk2)


def _extract(t, geom):
    N, _, Co = t.shape
    Ho, Wpx, Wv, BM, _ = geom
    return t[:, BM:BM + Ho * Wpx].reshape(N, Ho, Wpx, Co)[:, :, :Wv]


def _w9(W):
    return jnp.transpose(W, (2, 3, 1, 0)).reshape(9, W.shape[1], W.shape[0])


@jax.jit
def kernel(x, W_enc1, g_enc1, b_enc1, W_enc2, g_enc2, b_enc2, W_enc3, g_enc3,
           b_enc3, W_enc4, g_enc4, b_enc4, W_dec1, g_dec1, b_dec1, W_dec2,
           g_dec2, b_dec2, W_dec3, g_dec3, b_dec3, W_dec4, g_dec4, b_dec4,
           W_out, b_out):
    N = x.shape[0]
    BM = 2048
    act = jnp.transpose(x, (0, 2, 3, 1))
    act = jnp.pad(act, ((0, 0), (0, 0), (0, 0), (0, 7)))
    enc = [(W_enc1, g_enc1, b_enc1), (W_enc2, g_enc2, b_enc2),
           (W_enc3, g_enc3, b_enc3), (W_enc4, g_enc4, b_enc4)]
    t = None
    for k, (Wc, g_, be_) in enumerate(enc):
        w9 = _w9(Wc)
        if k == 0:
            w9 = jnp.pad(w9, ((0, 0), (0, 7), (0, 0)))
        y, st, geom = _enc_conv(act, w9, BM)
        cnt = jnp.float32(N * geom[0] * geom[2])
        t = _bn_relu_framed(y, st, g_, be_, cnt, geom)
        if k < 3:
            act = _extract(t, geom)
    latent = jnp.transpose(_extract(t, geom), (0, 3, 1, 2))
    dec = [(W_dec1, g_dec1, b_dec1), (W_dec2, g_dec2, b_dec2),
           (W_dec3, g_dec3, b_dec3), (W_dec4, g_dec4, b_dec4)]
    for Wc, g_, be_ in dec:
        up, geom = _up_to_framed(t, geom, BM)
        y, st, geom = _framed_conv(up, _w9(Wc), geom)
        cnt = jnp.float32(N * geom[0] * geom[2])
        t = _bn_relu_framed(y, st, g_, be_, cnt, geom)
    w9o = jnp.pad(_w9(W_out), ((0, 0), (0, 0), (0, 7)))
    bias = jnp.broadcast_to(b_out[0], (8, 8)).astype(jnp.float32)
    y, _, geom = _framed_conv(t, w9o, geom, sig_bias=bias)
    Ho, Wpx, Wv, _, _ = geom
    out = y[:, :Ho * Wpx].reshape(N, Ho, Wpx, 8)[:, :, :Wv, 0:1]
    out = jnp.transpose(out, (0, 3, 1, 2))
    return (out, latent)
